# baseline (speedup 1.0000x reference)
import jax
import jax.numpy as jnp
from jax import lax
from jax.experimental import pallas as pl
from jax.experimental.pallas import tpu as pltpu
from jax.experimental.pallas import tpu_sc as plsc

_C = 16
_ROWS_PER_WORKER = 1024
_NCH = _ROWS_PER_WORKER // _C
_NC = 2
_NS = 16


def _sc_body(hs_ref, keys_ref, vals_ref, b0, b1, si0, si1, so0, so1):
    w = lax.axis_index("c") * _NS + lax.axis_index("s")
    batch = w // 4
    r = w % 4
    half = r % 2
    src0 = (batch * 2048 + half * 1023 + jnp.where(r >= 2, 1, 0)) * 2048
    dst0 = (batch * 2047 + half * 1023) * 2048
    ch = _C * 2048

    def run(out_ref):
        def cin(c, buf, sem):
            return pltpu.make_async_copy(
                hs_ref.at[pl.ds(src0 + c * ch, ch)], buf, sem)

        def cout(c, buf, sem):
            return pltpu.make_async_copy(
                buf, out_ref.at[pl.ds(dst0 + c * ch, ch)], sem)

        cin(0, b0, si0).start()
        cin(1, b1, si1).start()

        def body(g, carry):
            c0 = 2 * g
            cin(c0, b0, si0).wait()
            cout(c0, b0, so0).start()
            cin(c0 + 1, b1, si1).wait()
            cout(c0 + 1, b1, so1).start()
            cout(c0, b0, so0).wait()
            cout(c0 + 1, b1, so1).wait()

            @pl.when(g < _NCH // 2 - 1)
            def _():
                cin(c0 + 2, b0, si0).start()
                cin(c0 + 3, b1, si1).start()

            return carry

        lax.fori_loop(0, _NCH // 2, body, 0)

    @pl.when(r < 2)
    def _():
        run(keys_ref)

    @pl.when(r >= 2)
    def _():
        run(vals_ref)


def kernel(hidden_states, attention_mask):
    del attention_mask
    B, T, D = hidden_states.shape
    hs_flat = hidden_states.reshape(B * T * D)
    out_sds = jax.ShapeDtypeStruct((B * (T - 1) * D,), hidden_states.dtype)
    mesh = plsc.VectorSubcoreMesh(core_axis_name="c", subcore_axis_name="s")
    f = pl.kernel(
        _sc_body,
        out_type=[out_sds, out_sds],
        mesh=mesh,
        scratch_types=[
            pltpu.VMEM((_C * D,), hidden_states.dtype),
            pltpu.VMEM((_C * D,), hidden_states.dtype),
            pltpu.SemaphoreType.DMA,
            pltpu.SemaphoreType.DMA,
            pltpu.SemaphoreType.DMA,
            pltpu.SemaphoreType.DMA,
        ],
    )
    keys, vals = f(hs_flat)
    return (keys.reshape(B * (T - 1), D), vals.reshape(B * (T - 1), D))


# speedup vs baseline: 1.0168x; 1.0168x over previous
import jax
import jax.numpy as jnp
from jax import lax
from jax.experimental import pallas as pl
from jax.experimental.pallas import tpu as pltpu
from jax.experimental.pallas import tpu_sc as plsc

_C = 8
_NBUF = 4
_ROWS_PER_WORKER = 1024
_NCH = _ROWS_PER_WORKER // _C
_NC = 2
_NS = 16


def _sc_body(hs_ref, keys_ref, vals_ref, *rest):
    bufs = rest[:_NBUF]
    sins = rest[_NBUF:2 * _NBUF]
    souts = rest[2 * _NBUF:3 * _NBUF]
    w = lax.axis_index("c") * _NS + lax.axis_index("s")
    batch = w // 4
    r = w % 4
    half = r % 2
    src0 = (batch * 2048 + half * 1023 + jnp.where(r >= 2, 1, 0)) * 2048
    dst0 = (batch * 2047 + half * 1023) * 2048
    ch = _C * 2048

    def run(out_ref):
        def cin(c, k):
            return pltpu.make_async_copy(
                hs_ref.at[pl.ds(src0 + c * ch, ch)], bufs[k], sins[k])

        def cout(c, k):
            return pltpu.make_async_copy(
                bufs[k], out_ref.at[pl.ds(dst0 + c * ch, ch)], souts[k])

        cin(0, 0).start()
        cin(1, 1).start()

        def body(g, carry):
            for k in range(_NBUF):
                c = _NBUF * g + k
                d = (k + 2) % _NBUF
                cin(c, k).wait()
                cout(c, k).start()

                @pl.when(c >= 2)
                def _():
                    cout(jnp.maximum(c - 2, 0), d).wait()

                @pl.when(c + 2 < _NCH)
                def _():
                    cin(c + 2, d).start()

            return carry

        lax.fori_loop(0, _NCH // _NBUF, body, 0)
        cout(_NCH - 2, (_NCH - 2) % _NBUF).wait()
        cout(_NCH - 1, (_NCH - 1) % _NBUF).wait()

    @pl.when(r < 2)
    def _():
        run(keys_ref)

    @pl.when(r >= 2)
    def _():
        run(vals_ref)


def kernel(hidden_states, attention_mask):
    del attention_mask
    B, T, D = hidden_states.shape
    hs_flat = hidden_states.reshape(B * T * D)
    out_sds = jax.ShapeDtypeStruct((B * (T - 1) * D,), hidden_states.dtype)
    mesh = plsc.VectorSubcoreMesh(core_axis_name="c", subcore_axis_name="s")
    f = pl.kernel(
        _sc_body,
        out_type=[out_sds, out_sds],
        mesh=mesh,
        scratch_types=(
            [pltpu.VMEM((_C * D,), hidden_states.dtype)] * _NBUF
            + [pltpu.SemaphoreType.DMA] * (2 * _NBUF)
        ),
    )
    keys, vals = f(hs_flat)
    return (keys.reshape(B * (T - 1), D), vals.reshape(B * (T - 1), D))


# speedup vs baseline: 1.1015x; 1.0833x over previous
import jax
import jax.numpy as jnp
from jax import lax
from jax.experimental import pallas as pl
from jax.experimental.pallas import tpu as pltpu
from jax.experimental.pallas import tpu_sc as plsc

_C = 8
_NBUF = 4
_ROWS_PER_WORKER = 512
_NCH = _ROWS_PER_WORKER // _C
_NC = 2
_NS = 16
_S = 512


def _sc_vals_body(hs_ref, vals_ref, *rest):
    bufs = rest[:_NBUF]
    sins = rest[_NBUF:2 * _NBUF]
    souts = rest[2 * _NBUF:3 * _NBUF]
    w = lax.axis_index("c") * _NS + lax.axis_index("s")
    batch = w // 4
    q = w % 4
    start = jnp.minimum(q * _ROWS_PER_WORKER, 2047 - _ROWS_PER_WORKER)
    src0 = (batch * 2048 + start + 1) * 2048
    dst0 = (batch * 2047 + start) * 2048
    ch = _C * 2048

    def cin(c, k):
        return pltpu.make_async_copy(
            hs_ref.at[pl.ds(src0 + c * ch, ch)], bufs[k], sins[k])

    def cout(c, k):
        return pltpu.make_async_copy(
            bufs[k], vals_ref.at[pl.ds(dst0 + c * ch, ch)], souts[k])

    cin(0, 0).start()
    cin(1, 1).start()

    def body(g, carry):
        for k in range(_NBUF):
            c = _NBUF * g + k
            d = (k + 2) % _NBUF
            cin(c, k).wait()
            cout(c, k).start()

            @pl.when(c >= 2)
            def _():
                cout(jnp.maximum(c - 2, 0), d).wait()

            @pl.when(c + 2 < _NCH)
            def _():
                cin(c + 2, d).start()

        return carry

    lax.fori_loop(0, _NCH // _NBUF, body, 0)
    cout(_NCH - 2, (_NCH - 2) % _NBUF).wait()
    cout(_NCH - 1, (_NCH - 1) % _NBUF).wait()


def _tc_keys_body(a_ref, o_ref):
    o_ref[0] = a_ref[0]


def kernel(hidden_states, attention_mask):
    del attention_mask
    B, T, D = hidden_states.shape
    hs_flat = hidden_states.reshape(B * T * D)
    vals_sds = jax.ShapeDtypeStruct((B * (T - 1) * D,), hidden_states.dtype)
    mesh = plsc.VectorSubcoreMesh(core_axis_name="c", subcore_axis_name="s")
    vals = pl.kernel(
        _sc_vals_body,
        out_type=vals_sds,
        mesh=mesh,
        scratch_types=(
            [pltpu.VMEM((_C * D,), hidden_states.dtype)] * _NBUF
            + [pltpu.SemaphoreType.DMA] * (2 * _NBUF)
        ),
    )(hs_flat)

    keys = pl.pallas_call(
        _tc_keys_body,
        grid=(B, T // _S),
        in_specs=[pl.BlockSpec((1, _S, D), lambda b, j: (b, j, 0))],
        out_specs=pl.BlockSpec((1, _S, D), lambda b, j: (b, j, 0)),
        out_shape=jax.ShapeDtypeStruct((B, T - 1, D), hidden_states.dtype),
    )(hidden_states)

    return (keys.reshape(B * (T - 1), D), vals.reshape(B * (T - 1), D))


# speedup vs baseline: 1.9730x; 1.7912x over previous
import jax
import jax.numpy as jnp
from jax import lax
from jax.experimental import pallas as pl
from jax.experimental.pallas import tpu as pltpu
from jax.experimental.pallas import tpu_sc as plsc

_C = 16
_NBUF = 3
_ROWS_PER_WORKER = 512
_NCH = _ROWS_PER_WORKER // _C
_NS = 16
_S = 512


def _sc_vals_body(hs_ref, vals_ref, *rest):
    bufs = rest[:_NBUF]
    idxs = rest[_NBUF:2 * _NBUF]
    sins = rest[2 * _NBUF:3 * _NBUF]
    souts = rest[3 * _NBUF:4 * _NBUF]
    w = lax.axis_index("c") * _NS + lax.axis_index("s")
    nrows = vals_ref.shape[0]
    dst0 = jnp.minimum(w * _ROWS_PER_WORKER, nrows - _ROWS_PER_WORKER)

    def start_in(c, k):
        g = dst0 + c * _C + lax.iota(jnp.int32, _C)
        b = lax.shift_right_logical(g, 11)
        r = g + b - lax.shift_left(b, 11)
        idxs[k][...] = g + 1 + b + jnp.where(r >= 2047, 1, 0)
        return pltpu.async_copy(hs_ref.at[idxs[k]], bufs[k], sins[k])

    def win(c, k):
        return pltpu.make_async_copy(hs_ref.at[idxs[k]], bufs[k], sins[k])

    def cout(c, k):
        return pltpu.make_async_copy(
            bufs[k], vals_ref.at[pl.ds(dst0 + c * _C, _C)], souts[k])

    start_in(0, 0)
    start_in(1, 1)

    def body(g, carry):
        for k in range(_NBUF):
            c = _NBUF * g + k
            d = (k + 2) % _NBUF
            win(c, k).wait()
            cout(c, k).start()

            @pl.when(c >= 1)
            def _():
                cout(jnp.maximum(c - 1, 0), d).wait()

            @pl.when(c + 2 < _NCH)
            def _():
                start_in(c + 2, d)

        return carry

    lax.fori_loop(0, _NCH // _NBUF, body, 0)
    for c in (_NCH - 2, _NCH - 1):
        k = c % _NBUF
        win(c, k).wait()
        cout(c, k).start()
    for c in (_NCH - 3, _NCH - 2, _NCH - 1):
        cout(c, c % _NBUF).wait()


def _tc_keys_body(a_ref, o_ref):
    o_ref[0] = a_ref[0]


def kernel(hidden_states, attention_mask):
    del attention_mask
    B, T, D = hidden_states.shape
    hs2d = hidden_states.reshape(B * T, D)
    vals_sds = jax.ShapeDtypeStruct((B * (T - 1), D), hidden_states.dtype)
    mesh = plsc.VectorSubcoreMesh(core_axis_name="c", subcore_axis_name="s")
    vals = pl.kernel(
        _sc_vals_body,
        out_type=vals_sds,
        mesh=mesh,
        scratch_types=(
            [pltpu.VMEM((_C, D), hidden_states.dtype)] * _NBUF
            + [pltpu.VMEM((_C,), jnp.int32)] * _NBUF
            + [pltpu.SemaphoreType.DMA] * (2 * _NBUF)
        ),
    )(hs2d)

    keys = pl.pallas_call(
        _tc_keys_body,
        grid=(B, T // _S),
        in_specs=[pl.BlockSpec((1, _S, D), lambda b, j: (b, j, 0))],
        out_specs=pl.BlockSpec((1, _S, D), lambda b, j: (b, j, 0)),
        out_shape=jax.ShapeDtypeStruct((B, T - 1, D), hidden_states.dtype),
    )(hidden_states)

    return (keys.reshape(B * (T - 1), D), vals)
